# baseline (speedup 1.0000x reference)
import jax
import jax.numpy as jnp
from jax.experimental import pallas as pl
from jax.experimental.pallas import tpu as pltpu

EPS = 1e-5
NB = 8
H = 256
W = 256
ROWS = NB * H
R = 32
SPS = H // R
G = ROWS // R
GC = G // 2
RC = 8
NPIX = NB * H * W


def _celu(x):
    return jnp.maximum(x, jnp.exp(jnp.minimum(x, 0.0)) - 1.0)


def _inst_kernel(x_ref, o_ref):
    x = x_ref[0]
    n = H * W
    m = jnp.sum(x, axis=(1, 2)) / n
    d = x - m[:, None, None]
    var = jnp.sum(d * d, axis=(1, 2)) / (n - 1)
    inv = jax.lax.rsqrt(var)
    m8 = jnp.concatenate([m, jnp.zeros((5,), jnp.float32)])
    i8 = jnp.concatenate([inv, jnp.zeros((5,), jnp.float32)])
    o_ref[0] = jnp.concatenate([m8[None], i8[None], jnp.zeros((6, 8), jnp.float32)], axis=0)


def _inst_call(x_nchw):
    return pl.pallas_call(
        _inst_kernel,
        grid=(NB,),
        in_specs=[pl.BlockSpec((1, 3, H, W), lambda i: (i, 0, 0, 0))],
        out_specs=pl.BlockSpec((1, 8, 8), lambda i: (i, 0, 0)),
        out_shape=jax.ShapeDtypeStruct((NB, 8, 8), jnp.float32),
        compiler_params=pltpu.CompilerParams(dimension_semantics=("parallel",)),
        name="inst_stats",
    )(x_nchw)


def _prep_kernel(x_ref, o_ref):
    o_ref[...] = jnp.transpose(x_ref[0], (1, 2, 0))


def _prep_call(x_nchw):
    return pl.pallas_call(
        _prep_kernel,
        grid=(2, GC),
        in_specs=[pl.BlockSpec(
            (1, 3, R, W),
            lambda c, i: ((c * GC + i) // SPS, 0, (c * GC + i) % SPS, 0))],
        out_specs=pl.BlockSpec((R, W, 3), lambda c, i: (c * GC + i, 0, 0)),
        out_shape=jax.ShapeDtypeStruct((ROWS, W, 3), jnp.float32),
        compiler_params=pltpu.CompilerParams(
            dimension_semantics=("parallel", "arbitrary")),
        name="prep_rows",
    )(x_nchw)


def _pack_kernel(y_ref, o_ref):
    o_ref[0] = jnp.transpose(y_ref[...], (2, 0, 1))


def _pack_call(y_r):
    return pl.pallas_call(
        _pack_kernel,
        grid=(2, GC),
        in_specs=[pl.BlockSpec((R, W, 3), lambda c, i: (c * GC + i, 0, 0))],
        out_specs=pl.BlockSpec(
            (1, 3, R, W),
            lambda c, i: ((c * GC + i) // SPS, 0, (c * GC + i) % SPS, 0)),
        out_shape=jax.ShapeDtypeStruct((NB, 3, H, W), jnp.float32),
        compiler_params=pltpu.CompilerParams(
            dimension_semantics=("parallel", "arbitrary")),
        name="pack_nchw",
    )(y_r)


def _stage1_kernel(xm_ref, xt_ref, xb_ref, inst_ref, w_ref,
                   o_ref, st_ref, act_scr, x9_scr):
    c, i = pl.program_id(0), pl.program_id(1)
    s = c * GC + i
    n = s // SPS
    first = (s % SPS) == 0
    last = (s % SPS) == (SPS - 1)
    inst = inst_ref[n]
    m = inst[0, 0:3]
    inv = inst[1, 0:3]

    def tr(v):
        xn = (v - m[None, None, :]) * inv[None, None, :]
        return jnp.concatenate([v, xn], axis=-1)

    top = tr(xt_ref[...])
    act_scr[0:1] = jnp.where(first, jnp.zeros_like(top), top)
    act_scr[1:R + 1] = tr(xm_ref[...])
    bot = tr(xb_ref[...])
    act_scr[R + 1:R + 2] = jnp.where(last, jnp.zeros_like(bot), bot)

    z1 = jnp.zeros((R, 1, 6), jnp.float32)
    for kh in range(3):
        a = act_scr[kh:kh + R]
        base = (kh * 3) * 6
        x9_scr[:, 1:, base:base + 6] = a[:, :W - 1, :]
        x9_scr[:, 0:1, base:base + 6] = z1
        x9_scr[:, :, base + 6:base + 12] = a
        x9_scr[:, :W - 1, base + 12:base + 18] = a[:, 1:, :]
        x9_scr[:, W - 1:W, base + 12:base + 18] = z1

    s1 = jnp.zeros((64,), jnp.float32)
    s2 = jnp.zeros((64,), jnp.float32)
    for j in range(R // RC):
        lhs = x9_scr[j * RC:(j + 1) * RC].reshape(RC * W, 54)
        z = jnp.dot(lhs, w_ref[...], preferred_element_type=jnp.float32)
        o_ref[j * RC:(j + 1) * RC] = z.reshape(RC, W, 64).astype(o_ref.dtype)
        s1 = s1 + jnp.sum(z, axis=0)
        s2 = s2 + jnp.sum(z * z, axis=0)
    st_ref[0] = jnp.concatenate([s1[None], s2[None], jnp.zeros((6, 64), jnp.float32)], axis=0)


def _make_stage_kernel(skip, wact, cout, final):
    def kern(*refs):
        idx = 0
        xm, xt, xb = refs[idx], refs[idx + 1], refs[idx + 2]; idx += 3
        if skip is not None:
            sm, st_, sb = refs[idx], refs[idx + 1], refs[idx + 2]; idx += 3
        ss_ref = refs[idx]; idx += 1
        w_ref = refs[idx]; idx += 1
        if final:
            xres_ref = refs[idx]; idx += 1
        o_ref = refs[idx]; idx += 1
        if not final:
            stat_ref = refs[idx]; idx += 1
        if wact:
            a_ref = refs[idx]; idx += 1
        x3_scr = refs[idx]; idx += 1
        z_scr = refs[idx]; idx += 1

        c, i = pl.program_id(0), pl.program_id(1)
        s = c * GC + i
        first = (s % SPS) == 0
        last = (s % SPS) == (SPS - 1)
        sc = ss_ref[0]
        sh = ss_ref[1]

        if skip == 'bn':
            sc2 = ss_ref[2]
            sh2 = ss_ref[3]

            def tr(v, vs):
                return (_celu(v * sc[None, None, :] + sh[None, None, :])
                        + _celu(vs * sc2[None, None, :] + sh2[None, None, :]))
        elif skip == 'id':
            def tr(v, vs):
                return _celu(v * sc[None, None, :] + sh[None, None, :]) + vs
        else:
            def tr(v, vs):
                return _celu(v * sc[None, None, :] + sh[None, None, :])

        def put(rows, v):
            x3_scr[rows, 1:, 0:64] = v[:, :W - 1, :]
            x3_scr[rows, :, 64:128] = v
            x3_scr[rows, :W - 1, 128:192] = v[:, 1:, :]

        topv = tr(xt[...], st_[...] if skip else None)
        put(slice(0, 1), jnp.where(first, jnp.zeros_like(topv), topv))
        mainv = tr(xm[...], sm[...] if skip else None)
        put(slice(1, R + 1), mainv)
        botv = tr(xb[...], sb[...] if skip else None)
        put(slice(R + 1, R + 2), jnp.where(last, jnp.zeros_like(botv), botv))
        x3_scr[:, 0:1, 0:64] = jnp.zeros((R + 2, 1, 64), jnp.float32)
        x3_scr[:, W - 1:W, 128:192] = jnp.zeros((R + 2, 1, 64), jnp.float32)

        if wact:
            a_ref[...] = mainv

        r0 = 0
        for rc in ([7, 7, 7, 7, 6] if R == 32 else [6, 6, 6]):
            lhs = x3_scr[r0:r0 + rc].reshape(rc * W, 192)
            z_scr[r0:r0 + rc] = jnp.dot(
                lhs, w_ref[...], preferred_element_type=jnp.float32
            ).reshape(rc, W, 256)
            r0 += rc

        s1 = jnp.zeros((64,), jnp.float32)
        s2 = jnp.zeros((64,), jnp.float32)
        for j in range(R // RC):
            t0 = j * RC
            z = (z_scr[t0:t0 + RC, :, 0:cout]
                 + z_scr[t0 + 1:t0 + RC + 1, :, 64:64 + cout]
                 + z_scr[t0 + 2:t0 + RC + 2, :, 128:128 + cout])
            if final:
                b = ss_ref[2][0:cout]
                zc = _celu(z + b[None, None, :]) + xres_ref[t0:t0 + RC]
                o_ref[t0:t0 + RC] = zc
            else:
                o_ref[t0:t0 + RC] = z.astype(o_ref.dtype)
                s1 = s1 + jnp.sum(z, axis=(0, 1))
                s2 = s2 + jnp.sum(z * z, axis=(0, 1))
        if not final:
            stat_ref[0] = jnp.concatenate(
                [s1[None], s2[None], jnp.zeros((6, 64), jnp.float32)], axis=0)

    return kern


def _row_specs(cin):
    main = pl.BlockSpec((R, W, cin), lambda c, i: (c * GC + i, 0, 0))
    top = pl.BlockSpec((1, W, cin),
                       lambda c, i: (jnp.maximum((c * GC + i) * R - 1, 0), 0, 0))
    bot = pl.BlockSpec((1, W, cin),
                       lambda c, i: (jnp.minimum((c * GC + i) * R + R, ROWS - 1), 0, 0))
    return [main, top, bot]


def _stage_call(name, x_raw, ss, w3, skip=None, skip_arr=None, wact=False,
                cout=64, final=False, xres=None):
    in_specs = _row_specs(64)
    args = [x_raw, x_raw, x_raw]
    if skip is not None:
        in_specs += _row_specs(64)
        args += [skip_arr, skip_arr, skip_arr]
    in_specs.append(pl.BlockSpec((8, 64), lambda c, i: (0, 0)))
    args.append(ss)
    in_specs.append(pl.BlockSpec((192, 256), lambda c, i: (0, 0)))
    args.append(w3)
    if final:
        in_specs.append(pl.BlockSpec((R, W, 3), lambda c, i: (c * GC + i, 0, 0)))
        args.append(xres)

    out_shapes = [jax.ShapeDtypeStruct((ROWS, W, cout), jnp.float32)]
    out_specs = [pl.BlockSpec((R, W, cout), lambda c, i: (c * GC + i, 0, 0))]
    if not final:
        out_shapes.append(jax.ShapeDtypeStruct((G, 8, 64), jnp.float32))
        out_specs.append(pl.BlockSpec((1, 8, 64), lambda c, i: (c * GC + i, 0, 0)))
    if wact:
        out_shapes.append(jax.ShapeDtypeStruct((ROWS, W, 64), jnp.float32))
        out_specs.append(pl.BlockSpec((R, W, 64), lambda c, i: (c * GC + i, 0, 0)))

    kern = _make_stage_kernel(skip, wact, cout, final)
    flat_args = []
    for a in args:
        if isinstance(a, tuple):
            flat_args.extend(a)
        else:
            flat_args.append(a)
    return pl.pallas_call(
        kern,
        grid=(2, GC),
        in_specs=in_specs,
        out_specs=out_specs,
        out_shape=out_shapes,
        scratch_shapes=[
            pltpu.VMEM((R + 2, W, 192), jnp.float32),
            pltpu.VMEM((R + 2, W, 256), jnp.float32),
        ],
        compiler_params=pltpu.CompilerParams(
            dimension_semantics=("parallel", "arbitrary")),
        name=name,
    )(*flat_args)


def _stage1_call(x_r, inst, w9):
    return pl.pallas_call(
        _stage1_kernel,
        grid=(2, GC),
        in_specs=_row_specs(3) + [
            pl.BlockSpec((NB, 8, 8), lambda c, i: (0, 0, 0)),
            pl.BlockSpec((54, 64), lambda c, i: (0, 0)),
        ],
        out_specs=[
            pl.BlockSpec((R, W, 64), lambda c, i: (c * GC + i, 0, 0)),
            pl.BlockSpec((1, 8, 64), lambda c, i: (c * GC + i, 0, 0)),
        ],
        out_shape=[
            jax.ShapeDtypeStruct((ROWS, W, 64), jnp.float32),
            jax.ShapeDtypeStruct((G, 8, 64), jnp.float32),
        ],
        scratch_shapes=[
            pltpu.VMEM((R + 2, W, 6), jnp.float32),
            pltpu.VMEM((R, W, 54), jnp.float32),
        ],
        compiler_params=pltpu.CompilerParams(
            dimension_semantics=("parallel", "arbitrary")),
        name="stage1",
    )(x_r, x_r, x_r, inst, w9)


def _bn_ss(stats, g, be):
    s1 = jnp.sum(stats[:, 0, :], axis=0)
    s2 = jnp.sum(stats[:, 1, :], axis=0)
    m = s1 / NPIX
    var = s2 / NPIX - m * m
    sc = g * jax.lax.rsqrt(var + EPS)
    sh = be - m * sc
    return sc, sh


def _pack_ss(sc, sh, sc2=None, sh2=None):
    rows = [sc[None], sh[None]]
    if sc2 is not None:
        rows += [sc2[None], sh2[None]]
    pad = 8 - len(rows)
    rows.append(jnp.zeros((pad, 64), jnp.float32))
    return jnp.concatenate(rows, axis=0)


def kernel(x, w1, b1, g1, be1, bw, bb, bg, bbe, w2, b2):
    w9 = w1.transpose(2, 3, 1, 0).reshape(54, 64)

    def _wide(w, cout):
        t = w.transpose(2, 3, 1, 0).reshape(3, 192, cout)
        blocks = [jnp.pad(t[k], ((0, 0), (0, 64 - cout))) for k in range(3)]
        return jnp.concatenate(
            blocks + [jnp.zeros((192, 64), jnp.float32)], axis=1)

    wb = [_wide(bw[i, k], 64) for i in range(3) for k in range(2)]
    w3f = _wide(w2, 3)

    inst = _inst_call(x)
    x_r = _prep_call(x)
    c1, st1 = _stage1_call(x_r, inst, w9)
    sc1, sh1 = _bn_ss(st1, g1, be1)

    c2, st2, a0 = _stage_call("k1", c1, _pack_ss(sc1, sh1), wb[0], wact=True)
    sc2_, sh2_ = _bn_ss(st2, bg[0, 0], bbe[0, 0])
    c3, st3 = _stage_call("k2", c2, _pack_ss(sc2_, sh2_), wb[1])
    sc3_, sh3_ = _bn_ss(st3, bg[0, 1], bbe[0, 1])
    c4, st4, a1 = _stage_call("k3", c3, _pack_ss(sc3_, sh3_), wb[2],
                              skip='id', skip_arr=a0, wact=True)
    sc4_, sh4_ = _bn_ss(st4, bg[1, 0], bbe[1, 0])
    c5, st5 = _stage_call("k4", c4, _pack_ss(sc4_, sh4_), wb[3])
    sc5_, sh5_ = _bn_ss(st5, bg[1, 1], bbe[1, 1])
    c6, st6, a2 = _stage_call("k5", c5, _pack_ss(sc5_, sh5_), wb[4],
                              skip='id', skip_arr=a1, wact=True)
    sc6_, sh6_ = _bn_ss(st6, bg[2, 0], bbe[2, 0])
    c7, st7 = _stage_call("k6", c6, _pack_ss(sc6_, sh6_), wb[5])
    sc7_, sh7_ = _bn_ss(st7, bg[2, 1], bbe[2, 1])
    b2row = jnp.concatenate([b2, jnp.zeros((61,), jnp.float32)])[None]
    ssf = jnp.concatenate([sc7_[None], sh7_[None], b2row,
                           jnp.zeros((5, 64), jnp.float32)], axis=0)
    y_r = _stage_call("k7", c7, ssf, w3f, skip='id', skip_arr=a2,
                      cout=3, final=True, xres=x_r)
    y = y_r[0] if isinstance(y_r, (list, tuple)) else y_r
    return _pack_call(y)


# speedup vs baseline: 1.0331x; 1.0331x over previous
import jax
import jax.numpy as jnp
from jax.experimental import pallas as pl
from jax.experimental.pallas import tpu as pltpu

EPS = 1e-5
NB = 8
H = 256
W = 256
WP = W // 2
ROWS = NB * H
R = 32
SPS = H // R
G = ROWS // R
GC = G // 2
RC = 8
NPIX = NB * H * W


def _celu(x):
    return jnp.maximum(x, jnp.exp(jnp.minimum(x, 0.0)) - 1.0)


def _inst_kernel(x_ref, o_ref):
    x = x_ref[0]
    n = H * W
    m = jnp.sum(x, axis=(1, 2)) / n
    d = x - m[:, None, None]
    var = jnp.sum(d * d, axis=(1, 2)) / (n - 1)
    inv = jax.lax.rsqrt(var)
    m8 = jnp.concatenate([m, jnp.zeros((5,), jnp.float32)])
    i8 = jnp.concatenate([inv, jnp.zeros((5,), jnp.float32)])
    o_ref[0] = jnp.concatenate([m8[None], i8[None], jnp.zeros((6, 8), jnp.float32)], axis=0)


def _inst_call(x_nchw):
    return pl.pallas_call(
        _inst_kernel,
        grid=(NB,),
        in_specs=[pl.BlockSpec((1, 3, H, W), lambda i: (i, 0, 0, 0))],
        out_specs=pl.BlockSpec((1, 8, 8), lambda i: (i, 0, 0)),
        out_shape=jax.ShapeDtypeStruct((NB, 8, 8), jnp.float32),
        compiler_params=pltpu.CompilerParams(dimension_semantics=("parallel",)),
        name="inst_stats",
    )(x_nchw)


def _prep_kernel(x_ref, o_ref):
    o_ref[...] = jnp.transpose(x_ref[0], (1, 2, 0))


def _prep_call(x_nchw):
    return pl.pallas_call(
        _prep_kernel,
        grid=(2, GC),
        in_specs=[pl.BlockSpec(
            (1, 3, R, W),
            lambda c, i: ((c * GC + i) // SPS, 0, (c * GC + i) % SPS, 0))],
        out_specs=pl.BlockSpec((R, W, 3), lambda c, i: (c * GC + i, 0, 0)),
        out_shape=jax.ShapeDtypeStruct((ROWS, W, 3), jnp.float32),
        compiler_params=pltpu.CompilerParams(
            dimension_semantics=("parallel", "arbitrary")),
        name="prep_rows",
    )(x_nchw)


def _pack_kernel(y_ref, o_ref):
    o_ref[0] = jnp.transpose(y_ref[...], (2, 0, 1))


def _pack_call(y_r):
    return pl.pallas_call(
        _pack_kernel,
        grid=(2, GC),
        in_specs=[pl.BlockSpec((R, W, 3), lambda c, i: (c * GC + i, 0, 0))],
        out_specs=pl.BlockSpec(
            (1, 3, R, W),
            lambda c, i: ((c * GC + i) // SPS, 0, (c * GC + i) % SPS, 0)),
        out_shape=jax.ShapeDtypeStruct((NB, 3, H, W), jnp.float32),
        compiler_params=pltpu.CompilerParams(
            dimension_semantics=("parallel", "arbitrary")),
        name="pack_nchw",
    )(y_r)


def _stage1_kernel(xm_ref, xt_ref, xb_ref, inst_ref, w_ref,
                   o_ref, st_ref, act_scr, x9e_scr, x9o_scr):
    c, i = pl.program_id(0), pl.program_id(1)
    s = c * GC + i
    n = s // SPS
    first = (s % SPS) == 0
    last = (s % SPS) == (SPS - 1)
    inst = inst_ref[n]
    m6 = jnp.concatenate([inst[0, 0:3], inst[0, 0:3]])
    i6 = jnp.concatenate([inst[1, 0:3], inst[1, 0:3]])

    def tr(v):
        xn = (v - m6[None, None, :]) * i6[None, None, :]
        return jnp.concatenate(
            [v[:, :, 0:3], xn[:, :, 0:3], v[:, :, 3:6], xn[:, :, 3:6]], axis=-1)

    top = tr(xt_ref[...])
    act_scr[0:1] = jnp.where(first, jnp.zeros_like(top), top)
    act_scr[1:R + 1] = tr(xm_ref[...])
    bot = tr(xb_ref[...])
    act_scr[R + 1:R + 2] = jnp.where(last, jnp.zeros_like(bot), bot)

    z6 = jnp.zeros((R, 1, 6), jnp.float32)
    for kh in range(3):
        a = act_scr[kh:kh + R]
        e6 = a[:, :, 0:6]
        o6 = a[:, :, 6:12]
        base = kh * 18
        x9e_scr[:, 1:, base:base + 6] = o6[:, :WP - 1, :]
        x9e_scr[:, 0:1, base:base + 6] = z6
        x9e_scr[:, :, base + 6:base + 12] = e6
        x9e_scr[:, :, base + 12:base + 18] = o6
        x9o_scr[:, :, base:base + 6] = e6
        x9o_scr[:, :, base + 6:base + 12] = o6
        x9o_scr[:, :WP - 1, base + 12:base + 18] = e6[:, 1:, :]
        x9o_scr[:, WP - 1:WP, base + 12:base + 18] = z6

    s1 = jnp.zeros((128,), jnp.float32)
    s2 = jnp.zeros((128,), jnp.float32)
    for j in range(R // RC):
        ze = jnp.dot(x9e_scr[j * RC:(j + 1) * RC].reshape(RC * WP, 54),
                     w_ref[...], preferred_element_type=jnp.float32)
        zo = jnp.dot(x9o_scr[j * RC:(j + 1) * RC].reshape(RC * WP, 54),
                     w_ref[...], preferred_element_type=jnp.float32)
        out = jnp.concatenate(
            [ze.reshape(RC, WP, 64), zo.reshape(RC, WP, 64)], axis=-1)
        o_ref[j * RC:(j + 1) * RC] = out
        s1 = s1 + jnp.sum(out, axis=(0, 1))
        s2 = s2 + jnp.sum(out * out, axis=(0, 1))
    st_ref[0] = jnp.concatenate(
        [s1[None], s2[None], jnp.zeros((6, 128), jnp.float32)], axis=0)


def _stage1_call(x_p, inst, w9):
    return pl.pallas_call(
        _stage1_kernel,
        grid=(2, GC),
        in_specs=_row_specs(6) + [
            pl.BlockSpec((NB, 8, 8), lambda c, i: (0, 0, 0)),
            pl.BlockSpec((54, 64), lambda c, i: (0, 0)),
        ],
        out_specs=[
            pl.BlockSpec((R, WP, 128), lambda c, i: (c * GC + i, 0, 0)),
            pl.BlockSpec((1, 8, 128), lambda c, i: (c * GC + i, 0, 0)),
        ],
        out_shape=[
            jax.ShapeDtypeStruct((ROWS, WP, 128), jnp.float32),
            jax.ShapeDtypeStruct((G, 8, 128), jnp.float32),
        ],
        scratch_shapes=[
            pltpu.VMEM((R + 2, WP, 12), jnp.float32),
            pltpu.VMEM((R, WP, 54), jnp.float32),
            pltpu.VMEM((R, WP, 54), jnp.float32),
        ],
        compiler_params=pltpu.CompilerParams(
            dimension_semantics=("parallel", "arbitrary")),
        name="stage1",
    )(x_p, x_p, x_p, inst, w9)


def _make_stage_kernel(skip, wact, cout, final):
    def kern(*refs):
        idx = 0
        xm, xt, xb = refs[idx], refs[idx + 1], refs[idx + 2]; idx += 3
        if skip is not None:
            sm, st_, sb = refs[idx], refs[idx + 1], refs[idx + 2]; idx += 3
        ss_ref = refs[idx]; idx += 1
        w_ref = refs[idx]; idx += 1
        if final:
            xres_ref = refs[idx]; idx += 1
        o_ref = refs[idx]; idx += 1
        if not final:
            stat_ref = refs[idx]; idx += 1
        if wact:
            a_ref = refs[idx]; idx += 1
        x3e_scr = refs[idx]; idx += 1
        x3o_scr = refs[idx]; idx += 1
        ze_scr = refs[idx]; idx += 1
        zo_scr = refs[idx]; idx += 1

        c, i = pl.program_id(0), pl.program_id(1)
        s = c * GC + i
        first = (s % SPS) == 0
        last = (s % SPS) == (SPS - 1)
        sc = ss_ref[0]
        sh = ss_ref[1]

        if skip == 'id':
            def tr(v, vs):
                return _celu(v * sc[None, None, :] + sh[None, None, :]) + vs
        else:
            def tr(v, vs):
                return _celu(v * sc[None, None, :] + sh[None, None, :])

        def put(rows, v):
            e = v[:, :, 0:64]
            o = v[:, :, 64:128]
            x3e_scr[rows, 1:, 0:64] = o[:, :WP - 1, :]
            x3e_scr[rows, :, 64:128] = e
            x3e_scr[rows, :, 128:192] = o
            x3o_scr[rows, :, 0:64] = e
            x3o_scr[rows, :, 64:128] = o
            x3o_scr[rows, :WP - 1, 128:192] = e[:, 1:, :]

        topv = tr(xt[...], st_[...] if skip else None)
        put(slice(0, 1), jnp.where(first, jnp.zeros_like(topv), topv))
        mainv = tr(xm[...], sm[...] if skip else None)
        put(slice(1, R + 1), mainv)
        botv = tr(xb[...], sb[...] if skip else None)
        put(slice(R + 1, R + 2), jnp.where(last, jnp.zeros_like(botv), botv))
        ze1 = jnp.zeros((R + 2, 1, 64), jnp.float32)
        x3e_scr[:, 0:1, 0:64] = ze1
        x3o_scr[:, WP - 1:WP, 128:192] = ze1

        if wact:
            a_ref[...] = mainv

        r0 = 0
        for rc in [7, 7, 7, 7, 6]:
            lhe = x3e_scr[r0:r0 + rc].reshape(rc * WP, 192)
            ze_scr[r0:r0 + rc] = jnp.dot(
                lhe, w_ref[...], preferred_element_type=jnp.float32
            ).reshape(rc, WP, 256)
            lho = x3o_scr[r0:r0 + rc].reshape(rc * WP, 192)
            zo_scr[r0:r0 + rc] = jnp.dot(
                lho, w_ref[...], preferred_element_type=jnp.float32
            ).reshape(rc, WP, 256)
            r0 += rc

        s1 = jnp.zeros((128,), jnp.float32)
        s2 = jnp.zeros((128,), jnp.float32)
        for j in range(R // RC):
            t0 = j * RC
            zev = (ze_scr[t0:t0 + RC, :, 0:cout]
                   + ze_scr[t0 + 1:t0 + RC + 1, :, 64:64 + cout]
                   + ze_scr[t0 + 2:t0 + RC + 2, :, 128:128 + cout])
            zov = (zo_scr[t0:t0 + RC, :, 0:cout]
                   + zo_scr[t0 + 1:t0 + RC + 1, :, 64:64 + cout]
                   + zo_scr[t0 + 2:t0 + RC + 2, :, 128:128 + cout])
            z = jnp.concatenate([zev, zov], axis=-1)
            if final:
                b = ss_ref[2][0:2 * cout]
                zc = _celu(z + b[None, None, :]) + xres_ref[t0:t0 + RC]
                o_ref[t0:t0 + RC] = zc
            else:
                o_ref[t0:t0 + RC] = z
                s1 = s1 + jnp.sum(z, axis=(0, 1))
                s2 = s2 + jnp.sum(z * z, axis=(0, 1))
        if not final:
            stat_ref[0] = jnp.concatenate(
                [s1[None], s2[None], jnp.zeros((6, 128), jnp.float32)], axis=0)

    return kern


def _row_specs(cl):
    main = pl.BlockSpec((R, WP, cl), lambda c, i: (c * GC + i, 0, 0))
    top = pl.BlockSpec((1, WP, cl),
                       lambda c, i: (jnp.maximum((c * GC + i) * R - 1, 0), 0, 0))
    bot = pl.BlockSpec((1, WP, cl),
                       lambda c, i: (jnp.minimum((c * GC + i) * R + R, ROWS - 1), 0, 0))
    return [main, top, bot]


def _stage_call(name, x_raw, ss, w3, skip=None, skip_arr=None, wact=False,
                cout=64, final=False, xres=None):
    in_specs = _row_specs(128)
    args = [x_raw, x_raw, x_raw]
    if skip is not None:
        in_specs += _row_specs(128)
        args += [skip_arr, skip_arr, skip_arr]
    in_specs.append(pl.BlockSpec((8, 128), lambda c, i: (0, 0)))
    args.append(ss)
    in_specs.append(pl.BlockSpec((192, 256), lambda c, i: (0, 0)))
    args.append(w3)
    if final:
        in_specs.append(pl.BlockSpec((R, WP, 6), lambda c, i: (c * GC + i, 0, 0)))
        args.append(xres)

    out_shapes = [jax.ShapeDtypeStruct((ROWS, WP, 2 * cout), jnp.float32)]
    out_specs = [pl.BlockSpec((R, WP, 2 * cout), lambda c, i: (c * GC + i, 0, 0))]
    if not final:
        out_shapes.append(jax.ShapeDtypeStruct((G, 8, 128), jnp.float32))
        out_specs.append(pl.BlockSpec((1, 8, 128), lambda c, i: (c * GC + i, 0, 0)))
    if wact:
        out_shapes.append(jax.ShapeDtypeStruct((ROWS, WP, 128), jnp.float32))
        out_specs.append(pl.BlockSpec((R, WP, 128), lambda c, i: (c * GC + i, 0, 0)))

    kern = _make_stage_kernel(skip, wact, cout, final)
    return pl.pallas_call(
        kern,
        grid=(2, GC),
        in_specs=in_specs,
        out_specs=out_specs,
        out_shape=out_shapes,
        scratch_shapes=[
            pltpu.VMEM((R + 2, WP, 192), jnp.float32),
            pltpu.VMEM((R + 2, WP, 192), jnp.float32),
            pltpu.VMEM((R + 2, WP, 256), jnp.float32),
            pltpu.VMEM((R + 2, WP, 256), jnp.float32),
        ],
        compiler_params=pltpu.CompilerParams(
            dimension_semantics=("parallel", "arbitrary")),
        name=name,
    )(*args)


def _bn_ss(stats, g, be):
    s1p = jnp.sum(stats[:, 0, :], axis=0)
    s2p = jnp.sum(stats[:, 1, :], axis=0)
    s1 = s1p[0:64] + s1p[64:128]
    s2 = s2p[0:64] + s2p[64:128]
    m = s1 / NPIX
    var = s2 / NPIX - m * m
    sc = g * jax.lax.rsqrt(var + EPS)
    sh = be - m * sc
    return sc, sh


def _pack_ss(sc, sh):
    sc2 = jnp.concatenate([sc, sc])[None]
    sh2 = jnp.concatenate([sh, sh])[None]
    return jnp.concatenate([sc2, sh2, jnp.zeros((6, 128), jnp.float32)], axis=0)


def kernel(x, w1, b1, g1, be1, bw, bb, bg, bbe, w2, b2):
    w9 = w1.transpose(2, 3, 1, 0).reshape(54, 64)

    def _wide(w, cout):
        t = w.transpose(2, 3, 1, 0).reshape(3, 192, cout)
        blocks = [jnp.pad(t[k], ((0, 0), (0, 64 - cout))) for k in range(3)]
        return jnp.concatenate(
            blocks + [jnp.zeros((192, 64), jnp.float32)], axis=1)

    wb = [_wide(bw[i, k], 64) for i in range(3) for k in range(2)]
    w3f = _wide(w2, 3)

    inst = _inst_call(x)
    x_p = _prep_call(x).reshape(ROWS, WP, 6)
    c1, st1 = _stage1_call(x_p, inst, w9)
    sc1, sh1 = _bn_ss(st1, g1, be1)

    c2, st2, a0 = _stage_call("k1", c1, _pack_ss(sc1, sh1), wb[0], wact=True)
    sc2_, sh2_ = _bn_ss(st2, bg[0, 0], bbe[0, 0])
    c3, st3 = _stage_call("k2", c2, _pack_ss(sc2_, sh2_), wb[1])
    sc3_, sh3_ = _bn_ss(st3, bg[0, 1], bbe[0, 1])
    c4, st4, a1 = _stage_call("k3", c3, _pack_ss(sc3_, sh3_), wb[2],
                              skip='id', skip_arr=a0, wact=True)
    sc4_, sh4_ = _bn_ss(st4, bg[1, 0], bbe[1, 0])
    c5, st5 = _stage_call("k4", c4, _pack_ss(sc4_, sh4_), wb[3])
    sc5_, sh5_ = _bn_ss(st5, bg[1, 1], bbe[1, 1])
    c6, st6, a2 = _stage_call("k5", c5, _pack_ss(sc5_, sh5_), wb[4],
                              skip='id', skip_arr=a1, wact=True)
    sc6_, sh6_ = _bn_ss(st6, bg[2, 0], bbe[2, 0])
    c7, st7 = _stage_call("k6", c6, _pack_ss(sc6_, sh6_), wb[5])
    sc7_, sh7_ = _bn_ss(st7, bg[2, 1], bbe[2, 1])
    b6 = jnp.concatenate([b2, b2])
    ssf = jnp.concatenate(
        [jnp.concatenate([sc7_, sc7_])[None],
         jnp.concatenate([sh7_, sh7_])[None],
         jnp.concatenate([b6, jnp.zeros((122,), jnp.float32)])[None],
         jnp.zeros((5, 128), jnp.float32)], axis=0)
    y_p = _stage_call("k7", c7, ssf, w3f, skip='id', skip_arr=a2,
                      cout=3, final=True, xres=x_p)
    y = y_p[0] if isinstance(y_p, (list, tuple)) else y_p
    return _pack_call(y.reshape(ROWS, W, 3))
